# Initial kernel scaffold; baseline (speedup 1.0000x reference)
import jax
import jax.numpy as jnp
from jax.experimental import pallas as pl


def kernel(anchors, image_sizes, objectness_logits, bbox_deltas):
    raise NotImplementedError("write your pallas kernel here")


# speedup vs baseline: 35.9642x; 35.9642x over previous
import functools

import jax
import jax.numpy as jnp
from jax import lax
from jax.experimental import pallas as pl
from jax.experimental.pallas import tpu as pltpu
from jax.experimental.pallas import tpu_sc as plsc

B = 8
N = 20000
PRE_NMS_TOPK = 6000
POST_NMS_TOPK = 1000
NMS_THRESH = 0.7

NPAD = 32768
SROWS = NPAD // 128
K = 6144
KROWS = K // 128
T = 512
NB = K // T
OUTK = 1024
MAXKEY = 0x7FFFFFFF


def _sort_body(score_ref, key_ref, idx_ref):
    s = score_ref[0]
    bits = jax.lax.bitcast_convert_type(s, jnp.int32)
    neg_inf = s == -jnp.inf
    key = jnp.where(neg_inf, MAXKEY, 0x40000000 - bits)
    row = lax.broadcasted_iota(jnp.int32, (SROWS, 128), 0)
    lane = lax.broadcasted_iota(jnp.int32, (SROWS, 128), 1)
    idx = row * 128 + lane

    def partner(a, j):
        if j < 128:
            down = jnp.roll(a, -j, axis=1)
            up = jnp.roll(a, j, axis=1)
            bit0 = (lane & j) == 0
        else:
            m = j // 128
            down = jnp.roll(a, -m, axis=0)
            up = jnp.roll(a, m, axis=0)
            bit0 = (row & m) == 0
        return jnp.where(bit0, down, up), bit0

    for kk in [2 << i for i in range(15)]:
        for j in [kk >> (1 + i) for i in range(kk.bit_length() - 1)]:
            kp, isfirst = partner(key, j)
            ip, _ = partner(idx, j)
            if kk < 128:
                dirup = (lane & kk) == 0
            else:
                dirup = (row & (kk // 128)) == 0
            wantsmall = isfirst == dirup
            lt = (key < kp) | ((key == kp) & (idx < ip))
            takeself = lt == wantsmall
            key = jnp.where(takeself, key, kp)
            idx = jnp.where(takeself, idx, ip)

    key_ref[0] = key[:KROWS, :]
    idx_ref[0] = idx[:KROWS, :]


def _run_sort(scores_padded):
    return pl.pallas_call(
        _sort_body,
        grid=(B,),
        in_specs=[pl.BlockSpec((1, SROWS, 128), lambda b: (b, 0, 0))],
        out_specs=[
            pl.BlockSpec((1, KROWS, 128), lambda b: (b, 0, 0)),
            pl.BlockSpec((1, KROWS, 128), lambda b: (b, 0, 0)),
        ],
        out_shape=[
            jax.ShapeDtypeStruct((B, KROWS, 128), jnp.int32),
            jax.ShapeDtypeStruct((B, KROWS, 128), jnp.int32),
        ],
    )(scores_padded)


NBCH = (B * K) // (32 * 128)


def _sc_gather_boxes(x1f, y1f, x2f, y2f, gidx):
    mesh = plsc.VectorSubcoreMesh(core_axis_name="c", subcore_axis_name="s")
    oshape = jax.ShapeDtypeStruct((32, NBCH, 128), jnp.float32)

    @functools.partial(
        pl.kernel,
        mesh=mesh,
        out_type=[oshape, oshape, oshape, oshape],
        scratch_types=[
            pltpu.VMEM((NBCH, 128), jnp.int32),
            pltpu.VMEM((NBCH, 128), jnp.float32),
            pltpu.SemaphoreType.DMA,
        ],
    )
    def kern(x1h, y1h, x2h, y2h, gih, o1, o2, o3, o4, idx_v, buf_v, sem):
        wid = lax.axis_index("s") * 2 + lax.axis_index("c")
        pltpu.sync_copy(gih.at[wid], idx_v)
        for table, out in ((x1h, o1), (y1h, o2), (x2h, o3), (y2h, o4)):
            copies = []
            for c in range(NBCH):
                copies.append(
                    pltpu.async_copy(table.at[idx_v.at[c]], buf_v.at[c], sem))
            for cp in copies:
                cp.wait()
            pltpu.sync_copy(buf_v, out.at[wid])

    return kern(x1f, y1f, x2f, y2f, gidx)


def _iou_gt(ix1, iy1, ix2, iy2, iarea, jx1, jy1, jx2, jy2, jarea):
    mx1 = jnp.maximum(ix1, jx1)
    my1 = jnp.maximum(iy1, jy1)
    mx2 = jnp.minimum(ix2, jx2)
    my2 = jnp.minimum(iy2, jy2)
    inter = jnp.maximum(0.0, mx2 - mx1) * jnp.maximum(0.0, my2 - my1)
    union = iarea + jarea - inter
    iou = jnp.where(union > 0.0, inter / union, 0.0)
    return jnp.where(iou > NMS_THRESH, 1.0, 0.0).astype(jnp.float32)


def _nms_body(x1c_ref, y1c_ref, x2c_ref, y2c_ref,
              x1r_ref, y1r_ref, x2r_ref, y2r_ref,
              keyr_ref, sidxr_ref, out_ref, kept_ref):
    kept_ref[...] = jnp.zeros((1, K), jnp.float32)

    iot_i = lax.broadcasted_iota(jnp.int32, (T, T), 0)
    iot_j = lax.broadcasted_iota(jnp.int32, (T, T), 1)
    tri = jnp.where(iot_i < iot_j, 1.0, 0.0).astype(jnp.float32)

    def colset(p):
        ix1 = x1c_ref[0, pl.ds(p * T, T), :]
        iy1 = y1c_ref[0, pl.ds(p * T, T), :]
        ix2 = x2c_ref[0, pl.ds(p * T, T), :]
        iy2 = y2c_ref[0, pl.ds(p * T, T), :]
        return ix1, iy1, ix2, iy2, (ix2 - ix1) * (iy2 - iy1)

    def rowset(r):
        jx1 = x1r_ref[0, :, pl.ds(r * T, T)]
        jy1 = y1r_ref[0, :, pl.ds(r * T, T)]
        jx2 = x2r_ref[0, :, pl.ds(r * T, T)]
        jy2 = y2r_ref[0, :, pl.ds(r * T, T)]
        return jx1, jy1, jx2, jy2, (jx2 - jx1) * (jy2 - jy1)

    def process_block(r, count):
        jset = rowset(r)
        pos_blk = r * T + lax.broadcasted_iota(jnp.int32, (1, T), 1)
        base = jnp.where(
            (keyr_ref[0, :, pl.ds(r * T, T)] != MAXKEY)
            & (pos_blk < PRE_NMS_TOPK), 1.0, 0.0).astype(jnp.float32)

        def cross(p, supin):
            def do(_):
                A = _iou_gt(*colset(p), *jset)
                kp = kept_ref[:, pl.ds(p * T, T)]
                return supin + jax.lax.dot_general(
                    kp, A, (((1,), (0,)), ((), ())),
                    preferred_element_type=jnp.float32)
            return lax.cond(p < r, do, lambda _: supin, 0)

        supin = lax.fori_loop(0, NB, cross, jnp.zeros((1, T), jnp.float32))
        base = jnp.where(supin > 0.5, 0.0, base)

        A_self = _iou_gt(*colset(r), *jset) * tri

        def fix_cond(carry):
            return carry[1]

        def fix_body(carry):
            kcur, _ = carry
            s = jax.lax.dot_general(kcur, A_self, (((1,), (0,)), ((), ())),
                                    preferred_element_type=jnp.float32)
            knew = jnp.where(s > 0.5, 0.0, base)
            return knew, jnp.any(knew != kcur)

        kfin, _ = lax.while_loop(fix_cond, fix_body,
                                 (base, jnp.bool_(True)))
        kept_ref[:, pl.ds(r * T, T)] = kfin
        return count + jnp.sum(kfin)

    def block_step(r, count):
        return lax.cond(count < float(POST_NMS_TOPK),
                        lambda c: process_block(r, c),
                        lambda c: c, count)

    total = lax.fori_loop(0, NB, block_step, jnp.float32(0.0))

    kept = kept_ref[...]
    cum = kept
    lanes = lax.broadcasted_iota(jnp.int32, (1, K), 1)
    for s in [1 << i for i in range(13)]:
        shifted = jnp.roll(cum, s, axis=1)
        cum = cum + jnp.where(lanes >= s, shifted, 0.0)
    rank = cum - kept

    payload_rows = [x1r_ref[0], y1r_ref[0], x2r_ref[0], y2r_ref[0],
                    sidxr_ref[0]]

    io0 = lax.broadcasted_iota(jnp.int32, (OUTK, OUTK), 0).astype(jnp.float32)
    accs = [jnp.zeros((OUTK, 1), jnp.float32) for _ in range(5)]
    for q in range(K // OUTK):
        lo, hi = q * OUTK, (q + 1) * OUTK
        S = jnp.where((rank[:, lo:hi] == io0) & (kept[:, lo:hi] > 0.5),
                      1.0, 0.0).astype(jnp.float32)
        for c in range(5):
            accs[c] = accs[c] + jnp.sum(
                S * payload_rows[c][:, lo:hi], axis=1, keepdims=True)

    io_out = lax.broadcasted_iota(jnp.int32, (OUTK, 1), 0).astype(jnp.float32)
    validc = jnp.where(io_out < total, 1.0, 0.0)
    out_ref[0] = jnp.concatenate(accs + [validc], axis=1)


def _run_nms(cols, rows, keyr, sidxr):
    x1c, y1c, x2c, y2c = cols
    x1r, y1r, x2r, y2r = rows
    cspec = pl.BlockSpec((1, K, 1), lambda b: (b, 0, 0))
    rspec = pl.BlockSpec((1, 1, K), lambda b: (b, 0, 0))
    return pl.pallas_call(
        _nms_body,
        grid=(B,),
        in_specs=[cspec, cspec, cspec, cspec,
                  rspec, rspec, rspec, rspec, rspec, rspec],
        out_specs=pl.BlockSpec((1, OUTK, 6), lambda b: (b, 0, 0)),
        out_shape=jax.ShapeDtypeStruct((B, OUTK, 6), jnp.float32),
        scratch_shapes=[pltpu.VMEM((1, K), jnp.float32)],
    )(x1c, y1c, x2c, y2c, x1r, y1r, x2r, y2r, keyr, sidxr)


NOCH = (B * OUTK) // (32 * 128)


def _sc_gather_outputs(lgf, d0f, d1f, d2f, d3f, gidx, validf):
    mesh = plsc.VectorSubcoreMesh(core_axis_name="c", subcore_axis_name="s")
    oshape = jax.ShapeDtypeStruct((32, NOCH, 128), jnp.float32)

    @functools.partial(
        pl.kernel,
        mesh=mesh,
        out_type=[oshape] * 5,
        scratch_types=[
            pltpu.VMEM((NOCH, 128), jnp.int32),
            pltpu.VMEM((NOCH, 128), jnp.float32),
            pltpu.VMEM((NOCH, 128), jnp.float32),
            pltpu.SemaphoreType.DMA,
        ],
    )
    def kern(lgh, d0h, d1h, d2h, d3h, gih, vh,
             ol, o0, o1, o2, o3, idx_v, val_v, buf_v, sem):
        wid = lax.axis_index("s") * 2 + lax.axis_index("c")
        pltpu.sync_copy(gih.at[wid], idx_v)
        pltpu.sync_copy(vh.at[wid], val_v)
        for table, out, fill in ((lgh, ol, 1e-08), (d0h, o0, 0.0),
                                 (d1h, o1, 0.0), (d2h, o2, 0.0),
                                 (d3h, o3, 0.0)):
            copies = []
            for c in range(NOCH):
                copies.append(
                    pltpu.async_copy(table.at[idx_v.at[c]], buf_v.at[c], sem))
            for cp in copies:
                cp.wait()
            for c in range(NOCH):
                for s in range(8):
                    sl = pl.ds(s * 16, 16)
                    v = buf_v[c, sl]
                    m = val_v[c, sl] > 0.5
                    buf_v[c, sl] = jnp.where(m, v, jnp.float32(fill))
            pltpu.sync_copy(buf_v, out.at[wid])

    return kern(lgf, d0f, d1f, d2f, d3f, gidx, validf)


def _decode_scores(anchors, image_sizes, objectness_logits, bbox_deltas):
    a_w = anchors[..., 2] - anchors[..., 0]
    a_h = anchors[..., 3] - anchors[..., 1]
    a_x = anchors[..., 0] + 0.5 * a_w
    a_y = anchors[..., 1] + 0.5 * a_h
    t_x = bbox_deltas[..., 0]
    t_y = bbox_deltas[..., 1]
    t_w = bbox_deltas[..., 2]
    t_h = bbox_deltas[..., 3]
    p_x = a_x + t_x * a_w
    p_y = a_y + t_y * a_h
    p_w = a_w * jnp.exp(jnp.clip(t_w, -10.0, 10.0))
    p_h = a_h * jnp.exp(jnp.clip(t_h, -10.0, 10.0))
    x1 = p_x - 0.5 * p_w
    y1 = p_y - 0.5 * p_h
    x2 = p_x + 0.5 * p_w
    y2 = p_y + 0.5 * p_h
    height = image_sizes[:, 0].astype(jnp.float32)[:, None]
    width = image_sizes[:, 1].astype(jnp.float32)[:, None]
    x1 = jnp.clip(x1, 0.0, width - 1.0)
    y1 = jnp.clip(y1, 0.0, height - 1.0)
    x2 = jnp.clip(x2, 0.0, width - 1.0)
    y2 = jnp.clip(y2, 0.0, height - 1.0)
    ws = x2 - x1
    hs = y2 - y1
    valid = (ws >= 0.0) & (hs >= 0.0)
    fg = jax.nn.sigmoid(objectness_logits[..., 0])
    fgm = jnp.where(valid, fg, -jnp.inf)
    return x1, y1, x2, y2, fgm


def kernel(anchors, image_sizes, objectness_logits, bbox_deltas):
    x1, y1, x2, y2, fgm = _decode_scores(
        anchors, image_sizes, objectness_logits, bbox_deltas)

    pad = jnp.full((B, NPAD - N), -jnp.inf, jnp.float32)
    scores_padded = jnp.concatenate([fgm, pad], axis=1).reshape(B, SROWS, 128)

    skey, sidx = _run_sort(scores_padded)
    skey = skey.reshape(B, K)
    sidx = sidx.reshape(B, K)

    img = jnp.arange(B, dtype=jnp.int32)[:, None]
    gidx1 = (img * N + jnp.minimum(sidx, N - 1)).reshape(32, NBCH, 128)
    bx1, by1, bx2, by2 = _sc_gather_boxes(
        x1.reshape(-1), y1.reshape(-1), x2.reshape(-1), y2.reshape(-1), gidx1)
    bx1 = bx1.reshape(B, K)
    by1 = by1.reshape(B, K)
    bx2 = bx2.reshape(B, K)
    by2 = by2.reshape(B, K)

    cols = tuple(a.reshape(B, K, 1) for a in (bx1, by1, bx2, by2))
    rows = tuple(a.reshape(B, 1, K) for a in (bx1, by1, bx2, by2))
    keyr = skey.reshape(B, 1, K)
    sidxr = sidx.astype(jnp.float32).reshape(B, 1, K)

    out6 = _run_nms(cols, rows, keyr, sidxr)

    props = out6[..., 0:4]
    oidx = out6[..., 4].astype(jnp.int32)
    validf = out6[..., 5]

    gidx2 = (img * N + oidx).reshape(32, NOCH, 128)
    vre = validf.reshape(32, NOCH, 128)
    lg, d0, d1, d2, d3 = _sc_gather_outputs(
        objectness_logits[..., 0].reshape(-1),
        bbox_deltas[..., 0].reshape(-1), bbox_deltas[..., 1].reshape(-1),
        bbox_deltas[..., 2].reshape(-1), bbox_deltas[..., 3].reshape(-1),
        gidx2, vre)

    props_out = (props * validf[..., None])[:, :POST_NMS_TOPK, :]
    logits_out = lg.reshape(B, OUTK, 1)[:, :POST_NMS_TOPK, :]
    deltas_out = jnp.stack(
        [d.reshape(B, OUTK) for d in (d0, d1, d2, d3)],
        axis=-1)[:, :POST_NMS_TOPK, :]
    validk = validf[:, :POST_NMS_TOPK] > 0.5
    return props_out, logits_out, deltas_out, validk


# speedup vs baseline: 38.1795x; 1.0616x over previous
import functools

import jax
import jax.numpy as jnp
from jax import lax
from jax.experimental import pallas as pl
from jax.experimental.pallas import tpu as pltpu
from jax.experimental.pallas import tpu_sc as plsc

B = 8
N = 20000
PRE_NMS_TOPK = 6000
POST_NMS_TOPK = 1000
NMS_THRESH = 0.7

NPAD = 32768
SROWS = NPAD // 128
K = 6144
KROWS = K // 128
T = 512
NB = K // T
OUTK = 1024
MAXKEY = 0x7FFFFFFF


GROWS = B * SROWS


def _sort_body(score_ref, key_ref, idx_ref):
    s = score_ref[...]
    bits = jax.lax.bitcast_convert_type(s, jnp.int32)
    neg_inf = s == -jnp.inf
    key = jnp.where(neg_inf, MAXKEY, 0x40000000 - bits)
    grow = lax.broadcasted_iota(jnp.int32, (GROWS, 128), 0)
    row = grow & (SROWS - 1)
    lane = lax.broadcasted_iota(jnp.int32, (GROWS, 128), 1)
    idx = row * 128 + lane

    def partner(a, j):
        if j < 128:
            down = jnp.roll(a, -j, axis=1)
            up = jnp.roll(a, j, axis=1)
            bit0 = (lane & j) == 0
        else:
            m = j // 128
            down = jnp.roll(a, -m, axis=0)
            up = jnp.roll(a, m, axis=0)
            bit0 = (row & m) == 0
        return jnp.where(bit0, down, up), bit0

    for kk in [2 << i for i in range(15)]:
        for j in [kk >> (1 + i) for i in range(kk.bit_length() - 1)]:
            kp, isfirst = partner(key, j)
            ip, _ = partner(idx, j)
            if kk < 128:
                dirup = (lane & kk) == 0
            else:
                dirup = (row & (kk // 128)) == 0
            wantsmall = isfirst == dirup
            lt = (key < kp) | ((key == kp) & (idx < ip))
            takeself = lt == wantsmall
            key = jnp.where(takeself, key, kp)
            idx = jnp.where(takeself, idx, ip)

    for b in range(B):
        key_ref[b] = key[b * SROWS:b * SROWS + KROWS, :]
        idx_ref[b] = idx[b * SROWS:b * SROWS + KROWS, :]


def _run_sort(scores_padded):
    return pl.pallas_call(
        _sort_body,
        out_shape=[
            jax.ShapeDtypeStruct((B, KROWS, 128), jnp.int32),
            jax.ShapeDtypeStruct((B, KROWS, 128), jnp.int32),
        ],
    )(scores_padded)


NBCH = (B * K) // (32 * 128)


def _sc_gather_boxes(x1f, y1f, x2f, y2f, gidx):
    mesh = plsc.VectorSubcoreMesh(core_axis_name="c", subcore_axis_name="s")
    oshape = jax.ShapeDtypeStruct((32, NBCH, 128), jnp.float32)

    @functools.partial(
        pl.kernel,
        mesh=mesh,
        out_type=[oshape, oshape, oshape, oshape],
        scratch_types=[
            pltpu.VMEM((NBCH, 128), jnp.int32),
            pltpu.VMEM((NBCH, 128), jnp.float32),
            pltpu.SemaphoreType.DMA,
        ],
    )
    def kern(x1h, y1h, x2h, y2h, gih, o1, o2, o3, o4, idx_v, buf_v, sem):
        wid = lax.axis_index("s") * 2 + lax.axis_index("c")
        pltpu.sync_copy(gih.at[wid], idx_v)
        for table, out in ((x1h, o1), (y1h, o2), (x2h, o3), (y2h, o4)):
            copies = []
            for c in range(NBCH):
                copies.append(
                    pltpu.async_copy(table.at[idx_v.at[c]], buf_v.at[c], sem))
            for cp in copies:
                cp.wait()
            pltpu.sync_copy(buf_v, out.at[wid])

    return kern(x1f, y1f, x2f, y2f, gidx)


def _iou_gt(ix1, iy1, ix2, iy2, iarea, jx1, jy1, jx2, jy2, jarea):
    mx1 = jnp.maximum(ix1, jx1)
    my1 = jnp.maximum(iy1, jy1)
    mx2 = jnp.minimum(ix2, jx2)
    my2 = jnp.minimum(iy2, jy2)
    inter = jnp.maximum(0.0, mx2 - mx1) * jnp.maximum(0.0, my2 - my1)
    union = iarea + jarea - inter
    iou = jnp.where(union > 0.0, inter / union, 0.0)
    return jnp.where(iou > NMS_THRESH, 1.0, 0.0).astype(jnp.float32)


def _nms_body(x1c_ref, y1c_ref, x2c_ref, y2c_ref,
              x1r_ref, y1r_ref, x2r_ref, y2r_ref,
              keyr_ref, sidxr_ref, out_ref, kept_ref):
    kept_ref[...] = jnp.zeros((1, K), jnp.float32)

    iot_i = lax.broadcasted_iota(jnp.int32, (T, T), 0)
    iot_j = lax.broadcasted_iota(jnp.int32, (T, T), 1)
    tri = jnp.where(iot_i < iot_j, 1.0, 0.0).astype(jnp.float32)

    def colset(p):
        ix1 = x1c_ref[0, pl.ds(p * T, T), :]
        iy1 = y1c_ref[0, pl.ds(p * T, T), :]
        ix2 = x2c_ref[0, pl.ds(p * T, T), :]
        iy2 = y2c_ref[0, pl.ds(p * T, T), :]
        return ix1, iy1, ix2, iy2, (ix2 - ix1) * (iy2 - iy1)

    def rowset(r):
        jx1 = x1r_ref[0, :, pl.ds(r * T, T)]
        jy1 = y1r_ref[0, :, pl.ds(r * T, T)]
        jx2 = x2r_ref[0, :, pl.ds(r * T, T)]
        jy2 = y2r_ref[0, :, pl.ds(r * T, T)]
        return jx1, jy1, jx2, jy2, (jx2 - jx1) * (jy2 - jy1)

    def process_block(r, count):
        jset = rowset(r)
        pos_blk = r * T + lax.broadcasted_iota(jnp.int32, (1, T), 1)
        base = jnp.where(
            (keyr_ref[0, :, pl.ds(r * T, T)] != MAXKEY)
            & (pos_blk < PRE_NMS_TOPK), 1.0, 0.0).astype(jnp.float32)

        def cross(p, supin):
            def do(_):
                A = _iou_gt(*colset(p), *jset)
                kp = kept_ref[:, pl.ds(p * T, T)]
                return supin + jax.lax.dot_general(
                    kp, A, (((1,), (0,)), ((), ())),
                    preferred_element_type=jnp.float32)
            return lax.cond(p < r, do, lambda _: supin, 0)

        supin = lax.fori_loop(0, NB, cross, jnp.zeros((1, T), jnp.float32))
        base = jnp.where(supin > 0.5, 0.0, base)

        A_self = _iou_gt(*colset(r), *jset) * tri

        def fix_cond(carry):
            return carry[1]

        def fix_body(carry):
            kcur, _ = carry
            s = jax.lax.dot_general(kcur, A_self, (((1,), (0,)), ((), ())),
                                    preferred_element_type=jnp.float32)
            knew = jnp.where(s > 0.5, 0.0, base)
            return knew, jnp.any(knew != kcur)

        kfin, _ = lax.while_loop(fix_cond, fix_body,
                                 (base, jnp.bool_(True)))
        kept_ref[:, pl.ds(r * T, T)] = kfin
        return count + jnp.sum(kfin)

    def block_step(r, count):
        return lax.cond(count < float(POST_NMS_TOPK),
                        lambda c: process_block(r, c),
                        lambda c: c, count)

    total = lax.fori_loop(0, NB, block_step, jnp.float32(0.0))

    kept = kept_ref[...]
    cum = kept
    lanes = lax.broadcasted_iota(jnp.int32, (1, K), 1)
    for s in [1 << i for i in range(13)]:
        shifted = jnp.roll(cum, s, axis=1)
        cum = cum + jnp.where(lanes >= s, shifted, 0.0)
    rank = cum - kept

    payload_rows = [x1r_ref[0], y1r_ref[0], x2r_ref[0], y2r_ref[0],
                    sidxr_ref[0]]

    io0 = lax.broadcasted_iota(jnp.int32, (OUTK, OUTK), 0).astype(jnp.float32)
    accs = [jnp.zeros((OUTK, 1), jnp.float32) for _ in range(5)]
    for q in range(K // OUTK):
        lo, hi = q * OUTK, (q + 1) * OUTK
        S = jnp.where((rank[:, lo:hi] == io0) & (kept[:, lo:hi] > 0.5),
                      1.0, 0.0).astype(jnp.float32)
        for c in range(5):
            accs[c] = accs[c] + jnp.sum(
                S * payload_rows[c][:, lo:hi], axis=1, keepdims=True)

    io_out = lax.broadcasted_iota(jnp.int32, (OUTK, 1), 0).astype(jnp.float32)
    validc = jnp.where(io_out < total, 1.0, 0.0)
    out_ref[0] = jnp.concatenate(accs + [validc], axis=1)


def _run_nms(cols, rows, keyr, sidxr):
    x1c, y1c, x2c, y2c = cols
    x1r, y1r, x2r, y2r = rows
    cspec = pl.BlockSpec((1, K, 1), lambda b: (b, 0, 0))
    rspec = pl.BlockSpec((1, 1, K), lambda b: (b, 0, 0))
    return pl.pallas_call(
        _nms_body,
        grid=(B,),
        in_specs=[cspec, cspec, cspec, cspec,
                  rspec, rspec, rspec, rspec, rspec, rspec],
        out_specs=pl.BlockSpec((1, OUTK, 6), lambda b: (b, 0, 0)),
        out_shape=jax.ShapeDtypeStruct((B, OUTK, 6), jnp.float32),
        scratch_shapes=[pltpu.VMEM((1, K), jnp.float32)],
    )(x1c, y1c, x2c, y2c, x1r, y1r, x2r, y2r, keyr, sidxr)


NOCH = (B * OUTK) // (32 * 128)


def _sc_gather_outputs(lgf, d0f, d1f, d2f, d3f, gidx, validf):
    mesh = plsc.VectorSubcoreMesh(core_axis_name="c", subcore_axis_name="s")
    oshape = jax.ShapeDtypeStruct((32, NOCH, 128), jnp.float32)

    @functools.partial(
        pl.kernel,
        mesh=mesh,
        out_type=[oshape] * 5,
        scratch_types=[
            pltpu.VMEM((NOCH, 128), jnp.int32),
            pltpu.VMEM((NOCH, 128), jnp.float32),
            pltpu.VMEM((NOCH, 128), jnp.float32),
            pltpu.SemaphoreType.DMA,
        ],
    )
    def kern(lgh, d0h, d1h, d2h, d3h, gih, vh,
             ol, o0, o1, o2, o3, idx_v, val_v, buf_v, sem):
        wid = lax.axis_index("s") * 2 + lax.axis_index("c")
        pltpu.sync_copy(gih.at[wid], idx_v)
        pltpu.sync_copy(vh.at[wid], val_v)
        for table, out, fill in ((lgh, ol, 1e-08), (d0h, o0, 0.0),
                                 (d1h, o1, 0.0), (d2h, o2, 0.0),
                                 (d3h, o3, 0.0)):
            copies = []
            for c in range(NOCH):
                copies.append(
                    pltpu.async_copy(table.at[idx_v.at[c]], buf_v.at[c], sem))
            for cp in copies:
                cp.wait()
            for c in range(NOCH):
                for s in range(8):
                    sl = pl.ds(s * 16, 16)
                    v = buf_v[c, sl]
                    m = val_v[c, sl] > 0.5
                    buf_v[c, sl] = jnp.where(m, v, jnp.float32(fill))
            pltpu.sync_copy(buf_v, out.at[wid])

    return kern(lgf, d0f, d1f, d2f, d3f, gidx, validf)


def _decode_scores(anchors, image_sizes, objectness_logits, bbox_deltas):
    a_w = anchors[..., 2] - anchors[..., 0]
    a_h = anchors[..., 3] - anchors[..., 1]
    a_x = anchors[..., 0] + 0.5 * a_w
    a_y = anchors[..., 1] + 0.5 * a_h
    t_x = bbox_deltas[..., 0]
    t_y = bbox_deltas[..., 1]
    t_w = bbox_deltas[..., 2]
    t_h = bbox_deltas[..., 3]
    p_x = a_x + t_x * a_w
    p_y = a_y + t_y * a_h
    p_w = a_w * jnp.exp(jnp.clip(t_w, -10.0, 10.0))
    p_h = a_h * jnp.exp(jnp.clip(t_h, -10.0, 10.0))
    x1 = p_x - 0.5 * p_w
    y1 = p_y - 0.5 * p_h
    x2 = p_x + 0.5 * p_w
    y2 = p_y + 0.5 * p_h
    height = image_sizes[:, 0].astype(jnp.float32)[:, None]
    width = image_sizes[:, 1].astype(jnp.float32)[:, None]
    x1 = jnp.clip(x1, 0.0, width - 1.0)
    y1 = jnp.clip(y1, 0.0, height - 1.0)
    x2 = jnp.clip(x2, 0.0, width - 1.0)
    y2 = jnp.clip(y2, 0.0, height - 1.0)
    ws = x2 - x1
    hs = y2 - y1
    valid = (ws >= 0.0) & (hs >= 0.0)
    fg = jax.nn.sigmoid(objectness_logits[..., 0])
    fgm = jnp.where(valid, fg, -jnp.inf)
    return x1, y1, x2, y2, fgm


def kernel(anchors, image_sizes, objectness_logits, bbox_deltas):
    x1, y1, x2, y2, fgm = _decode_scores(
        anchors, image_sizes, objectness_logits, bbox_deltas)

    pad = jnp.full((B, NPAD - N), -jnp.inf, jnp.float32)
    scores_padded = jnp.concatenate([fgm, pad], axis=1).reshape(GROWS, 128)

    skey, sidx = _run_sort(scores_padded)
    skey = skey.reshape(B, K)
    sidx = sidx.reshape(B, K)

    img = jnp.arange(B, dtype=jnp.int32)[:, None]
    gidx1 = (img * N + jnp.minimum(sidx, N - 1)).reshape(32, NBCH, 128)
    bx1, by1, bx2, by2 = _sc_gather_boxes(
        x1.reshape(-1), y1.reshape(-1), x2.reshape(-1), y2.reshape(-1), gidx1)
    bx1 = bx1.reshape(B, K)
    by1 = by1.reshape(B, K)
    bx2 = bx2.reshape(B, K)
    by2 = by2.reshape(B, K)

    cols = tuple(a.reshape(B, K, 1) for a in (bx1, by1, bx2, by2))
    rows = tuple(a.reshape(B, 1, K) for a in (bx1, by1, bx2, by2))
    keyr = skey.reshape(B, 1, K)
    sidxr = sidx.astype(jnp.float32).reshape(B, 1, K)

    out6 = _run_nms(cols, rows, keyr, sidxr)

    props = out6[..., 0:4]
    oidx = out6[..., 4].astype(jnp.int32)
    validf = out6[..., 5]

    gidx2 = (img * N + oidx).reshape(32, NOCH, 128)
    vre = validf.reshape(32, NOCH, 128)
    lg, d0, d1, d2, d3 = _sc_gather_outputs(
        objectness_logits[..., 0].reshape(-1),
        bbox_deltas[..., 0].reshape(-1), bbox_deltas[..., 1].reshape(-1),
        bbox_deltas[..., 2].reshape(-1), bbox_deltas[..., 3].reshape(-1),
        gidx2, vre)

    props_out = (props * validf[..., None])[:, :POST_NMS_TOPK, :]
    logits_out = lg.reshape(B, OUTK, 1)[:, :POST_NMS_TOPK, :]
    deltas_out = jnp.stack(
        [d.reshape(B, OUTK) for d in (d0, d1, d2, d3)],
        axis=-1)[:, :POST_NMS_TOPK, :]
    validk = validf[:, :POST_NMS_TOPK] > 0.5
    return props_out, logits_out, deltas_out, validk
